# Initial kernel scaffold; baseline (speedup 1.0000x reference)
import jax
import jax.numpy as jnp
from jax.experimental import pallas as pl


def kernel(t, gamma):
    raise NotImplementedError("write your pallas kernel here")


# speedup vs baseline: 4.5253x; 4.5253x over previous
import functools

import jax
import jax.numpy as jnp
from jax import lax
from jax.experimental import pallas as pl
from jax.experimental.pallas import tpu as pltpu
from jax.experimental.pallas import tpu_sc as plsc

_TIMESTEPS = 1000.0
_BATCH = 16384
_TABLE = 1001
_TABLE_PAD = 1008
_NC, _NS, _L = 2, 16, 16
_NW = _NC * _NS
_B_PER_W = _BATCH // _NW
_MAGIC = 12582912.0


def _body(t_hbm, g_hbm, out_hbm, t_v, g_v, o_v):
    wid = lax.axis_index("s") * _NC + lax.axis_index("c")
    base = wid * _B_PER_W
    pltpu.sync_copy(g_hbm, g_v)
    pltpu.sync_copy(t_hbm.at[pl.ds(base, _B_PER_W)], t_v)
    for i in range(_B_PER_W // _L):
        tv = t_v[pl.ds(i * _L, _L)]
        r = (tv * _TIMESTEPS + _MAGIC) - _MAGIC
        idx = r.astype(jnp.int32)
        o_v[pl.ds(i * _L, _L)] = plsc.load_gather(g_v, [idx])
    pltpu.sync_copy(o_v, out_hbm.at[pl.ds(base, _B_PER_W)])


@jax.jit
def kernel(t, gamma):
    g = jnp.pad(gamma, (0, _TABLE_PAD - _TABLE))
    mesh = plsc.VectorSubcoreMesh(core_axis_name="c", subcore_axis_name="s")
    f = functools.partial(
        pl.kernel,
        mesh=mesh,
        out_type=jax.ShapeDtypeStruct((_BATCH,), jnp.float32),
        scratch_types=[
            pltpu.VMEM((_B_PER_W,), jnp.float32),
            pltpu.VMEM((_TABLE_PAD,), jnp.float32),
            pltpu.VMEM((_B_PER_W,), jnp.float32),
        ],
        compiler_params=pltpu.CompilerParams(needs_layout_passes=False),
    )(_body)
    return f(t, g)


# speedup vs baseline: 4.6041x; 1.0174x over previous
import functools

import jax
import jax.numpy as jnp
from jax import lax
from jax.experimental import pallas as pl
from jax.experimental.pallas import tpu as pltpu
from jax.experimental.pallas import tpu_sc as plsc

_TIMESTEPS = 1000.0
_BATCH = 16384
_TABLE = 1001
_TABLE_PAD = 1008
_NC, _NS, _L = 2, 16, 16
_NW = _NC * _NS
_B_PER_W = _BATCH // _NW
_MAGIC = 12582912.0


def _body(t_hbm, g_hbm, out_hbm, t_v, g_v, o_v, sem_g, sem_t):
    wid = lax.axis_index("s") * _NC + lax.axis_index("c")
    base = wid * _B_PER_W
    cp_g = pltpu.async_copy(g_hbm, g_v, sem_g)
    cp_t = pltpu.async_copy(t_hbm.at[pl.ds(base, _B_PER_W)], t_v, sem_t)
    cp_g.wait()
    cp_t.wait()
    for i in range(_B_PER_W // _L):
        tv = t_v[pl.ds(i * _L, _L)]
        r = (tv * _TIMESTEPS + _MAGIC) - _MAGIC
        idx = r.astype(jnp.int32)
        o_v[pl.ds(i * _L, _L)] = plsc.load_gather(g_v, [idx])
    pltpu.sync_copy(o_v, out_hbm.at[pl.ds(base, _B_PER_W)])


@jax.jit
def kernel(t, gamma):
    mesh = plsc.VectorSubcoreMesh(core_axis_name="c", subcore_axis_name="s")
    f = functools.partial(
        pl.kernel,
        mesh=mesh,
        out_type=jax.ShapeDtypeStruct((_BATCH,), jnp.float32),
        scratch_types=[
            pltpu.VMEM((_B_PER_W,), jnp.float32),
            pltpu.VMEM((_TABLE,), jnp.float32),
            pltpu.VMEM((_B_PER_W,), jnp.float32),
            pltpu.SemaphoreType.DMA,
            pltpu.SemaphoreType.DMA,
        ],
        compiler_params=pltpu.CompilerParams(needs_layout_passes=False),
    )(_body)
    return f(t, gamma)


# speedup vs baseline: 4.6390x; 1.0076x over previous
import functools

import jax
import jax.numpy as jnp
from jax import lax
from jax.experimental import pallas as pl
from jax.experimental.pallas import tpu as pltpu
from jax.experimental.pallas import tpu_sc as plsc

_TIMESTEPS = 1000.0
_BATCH = 16384
_TABLE = 1001
_TABLE_PAD = 1008
_NC, _NS, _L = 2, 16, 16
_NW = _NC * _NS
_B_PER_W = _BATCH // _NW
_MAGIC = 12582912.0


def _body(t_hbm, g_hbm, out_hbm, t_v, g_v, o_v, sem_g, sem_t):
    wid = lax.axis_index("s") * _NC + lax.axis_index("c")
    base = wid * _B_PER_W
    cp_g = pltpu.async_copy(g_hbm, g_v, sem_g)
    cp_t = pltpu.async_copy(t_hbm.at[pl.ds(base, _B_PER_W)], t_v, sem_t)
    cp_g.wait()
    cp_t.wait()
    for i in range(_B_PER_W // _L):
        tv = t_v[pl.ds(i * _L, _L)]
        r = (tv * _TIMESTEPS + _MAGIC) - _MAGIC
        idx = r.astype(jnp.int32)
        o_v[pl.ds(i * _L, _L)] = plsc.load_gather(g_v, [idx])
    pltpu.sync_copy(o_v, out_hbm.at[pl.ds(base, _B_PER_W)])


@jax.jit
def kernel(t, gamma):
    mesh = plsc.VectorSubcoreMesh(core_axis_name="c", subcore_axis_name="s")
    f = functools.partial(
        pl.kernel,
        mesh=mesh,
        out_type=jax.ShapeDtypeStruct((_BATCH,), jnp.float32),
        scratch_types=[
            pltpu.VMEM((_B_PER_W,), jnp.float32),
            pltpu.VMEM((_TABLE,), jnp.float32),
            pltpu.VMEM((_B_PER_W,), jnp.float32),
            pltpu.SemaphoreType.DMA,
            pltpu.SemaphoreType.DMA,
        ],
        compiler_params=pltpu.CompilerParams(
            needs_layout_passes=False,
            skip_device_barrier=True,
            disable_bounds_checks=True,
            disable_semaphore_checks=True,
        ),
    )(_body)
    return f(t, gamma)


# speedup vs baseline: 4.9855x; 1.0747x over previous
import functools

import jax
import jax.numpy as jnp
from jax import lax
from jax.experimental import pallas as pl
from jax.experimental.pallas import tpu as pltpu
from jax.experimental.pallas import tpu_sc as plsc

_TIMESTEPS = 1000.0
_BATCH = 16384
_TABLE = 1001
_TABLE_PAD = 1008
_NC, _NS, _L = 1, 16, 16
_NW = _NC * _NS
_B_PER_W = _BATCH // _NW
_MAGIC = 12582912.0


def _body(t_hbm, g_hbm, out_hbm, t_v, g_v, o_v, sem_g, sem_t):
    wid = lax.axis_index("s") * _NC + lax.axis_index("c") if _NC > 1 else lax.axis_index("s")
    base = wid * _B_PER_W
    cp_g = pltpu.async_copy(g_hbm, g_v, sem_g)
    cp_t = pltpu.async_copy(t_hbm.at[pl.ds(base, _B_PER_W)], t_v, sem_t)
    cp_g.wait()
    cp_t.wait()
    for i in range(_B_PER_W // _L):
        tv = t_v[pl.ds(i * _L, _L)]
        r = (tv * _TIMESTEPS + _MAGIC) - _MAGIC
        idx = r.astype(jnp.int32)
        o_v[pl.ds(i * _L, _L)] = plsc.load_gather(g_v, [idx])
    pltpu.sync_copy(o_v, out_hbm.at[pl.ds(base, _B_PER_W)])


@jax.jit
def kernel(t, gamma):
    mesh = plsc.VectorSubcoreMesh(
        core_axis_name="c", subcore_axis_name="s", num_cores=_NC
    )
    f = functools.partial(
        pl.kernel,
        mesh=mesh,
        out_type=jax.ShapeDtypeStruct((_BATCH,), jnp.float32),
        scratch_types=[
            pltpu.VMEM((_B_PER_W,), jnp.float32),
            pltpu.VMEM((_TABLE,), jnp.float32),
            pltpu.VMEM((_B_PER_W,), jnp.float32),
            pltpu.SemaphoreType.DMA,
            pltpu.SemaphoreType.DMA,
        ],
        compiler_params=pltpu.CompilerParams(
            needs_layout_passes=False,
            skip_device_barrier=True,
            disable_bounds_checks=True,
            disable_semaphore_checks=True,
        ),
    )(_body)
    return f(t, gamma)


# speedup vs baseline: 5.0361x; 1.0101x over previous
import functools

import jax
import jax.numpy as jnp
from jax import lax
from jax.experimental import pallas as pl
from jax.experimental.pallas import tpu as pltpu
from jax.experimental.pallas import tpu_sc as plsc

_TIMESTEPS = 1000.0
_BATCH = 16384
_TABLE = 1001
_TABLE_PAD = 1008
_NC, _NS, _L = 1, 16, 16
_NW = _NC * _NS
_B_PER_W = _BATCH // _NW
_MAGIC = 12582912.0
_UNROLL = 8


def _body(t_hbm, g_hbm, out_hbm, t_v, g_v, o_v, sem_g, sem_t):
    wid = lax.axis_index("s") * _NC + lax.axis_index("c") if _NC > 1 else lax.axis_index("s")
    base = wid * _B_PER_W
    cp_g = pltpu.async_copy(g_hbm, g_v, sem_g)
    cp_t = pltpu.async_copy(t_hbm.at[pl.ds(base, _B_PER_W)], t_v, sem_t)
    cp_g.wait()
    cp_t.wait()
    def step(j, carry):
        base_j = j * (_UNROLL * _L)
        for u in range(_UNROLL):
            off = base_j + u * _L
            tv = t_v[pl.ds(off, _L)]
            r = (tv * _TIMESTEPS + _MAGIC) - _MAGIC
            idx = r.astype(jnp.int32)
            o_v[pl.ds(off, _L)] = plsc.load_gather(g_v, [idx])
        return carry

    lax.fori_loop(0, _B_PER_W // (_UNROLL * _L), step, 0)
    pltpu.sync_copy(o_v, out_hbm.at[pl.ds(base, _B_PER_W)])


@jax.jit
def kernel(t, gamma):
    mesh = plsc.VectorSubcoreMesh(
        core_axis_name="c", subcore_axis_name="s", num_cores=_NC
    )
    f = functools.partial(
        pl.kernel,
        mesh=mesh,
        out_type=jax.ShapeDtypeStruct((_BATCH,), jnp.float32),
        scratch_types=[
            pltpu.VMEM((_B_PER_W,), jnp.float32),
            pltpu.VMEM((_TABLE,), jnp.float32),
            pltpu.VMEM((_B_PER_W,), jnp.float32),
            pltpu.SemaphoreType.DMA,
            pltpu.SemaphoreType.DMA,
        ],
        compiler_params=pltpu.CompilerParams(
            needs_layout_passes=False,
            skip_device_barrier=True,
            disable_bounds_checks=True,
            disable_semaphore_checks=True,
        ),
    )(_body)
    return f(t, gamma)


# speedup vs baseline: 5.1112x; 1.0149x over previous
import functools

import jax
import jax.numpy as jnp
from jax import lax
from jax.experimental import pallas as pl
from jax.experimental.pallas import tpu as pltpu
from jax.experimental.pallas import tpu_sc as plsc

_TIMESTEPS = 1000.0
_BATCH = 16384
_TABLE = 1001
_TABLE_PAD = 1008
_NC, _NS, _L = 1, 16, 16
_NW = _NC * _NS
_B_PER_W = _BATCH // _NW
_MAGIC = 12582912.0
_UNROLL = 2


def _body(t_hbm, g_hbm, out_hbm, t_v, g_v, o_v, sem_g, sem_t):
    wid = lax.axis_index("s") * _NC + lax.axis_index("c") if _NC > 1 else lax.axis_index("s")
    base = wid * _B_PER_W
    cp_g = pltpu.async_copy(g_hbm, g_v, sem_g)
    cp_t = pltpu.async_copy(t_hbm.at[pl.ds(base, _B_PER_W)], t_v, sem_t)
    cp_g.wait()
    cp_t.wait()
    def step(j, carry):
        base_j = j * (_UNROLL * _L)
        for u in range(_UNROLL):
            off = base_j + u * _L
            tv = t_v[pl.ds(off, _L)]
            r = (tv * _TIMESTEPS + _MAGIC) - _MAGIC
            idx = r.astype(jnp.int32)
            o_v[pl.ds(off, _L)] = plsc.load_gather(g_v, [idx])
        return carry

    lax.fori_loop(0, _B_PER_W // (_UNROLL * _L), step, 0)
    pltpu.sync_copy(o_v, out_hbm.at[pl.ds(base, _B_PER_W)])


@jax.jit
def kernel(t, gamma):
    mesh = plsc.VectorSubcoreMesh(
        core_axis_name="c", subcore_axis_name="s", num_cores=_NC
    )
    f = functools.partial(
        pl.kernel,
        mesh=mesh,
        out_type=jax.ShapeDtypeStruct((_BATCH,), jnp.float32),
        scratch_types=[
            pltpu.VMEM((_B_PER_W,), jnp.float32),
            pltpu.VMEM((_TABLE,), jnp.float32),
            pltpu.VMEM((_B_PER_W,), jnp.float32),
            pltpu.SemaphoreType.DMA,
            pltpu.SemaphoreType.DMA,
        ],
        compiler_params=pltpu.CompilerParams(
            needs_layout_passes=False,
            skip_device_barrier=True,
            disable_bounds_checks=True,
            disable_semaphore_checks=True,
        ),
    )(_body)
    return f(t, gamma)


# speedup vs baseline: 5.1173x; 1.0012x over previous
import functools

import jax
import jax.numpy as jnp
from jax import lax
from jax.experimental import pallas as pl
from jax.experimental.pallas import tpu as pltpu
from jax.experimental.pallas import tpu_sc as plsc

_TIMESTEPS = 1000.0
_BATCH = 16384
_TABLE = 1001
_TABLE_PAD = 1008
_NC, _NS, _L = 1, 16, 16
_NW = _NC * _NS
_B_PER_W = _BATCH // _NW
_MAGIC = 12582912.0
_UNROLL = 1


def _body(t_hbm, g_hbm, out_hbm, t_v, g_v, o_v, sem_g, sem_t):
    wid = lax.axis_index("s") * _NC + lax.axis_index("c") if _NC > 1 else lax.axis_index("s")
    base = wid * _B_PER_W
    cp_g = pltpu.async_copy(g_hbm, g_v, sem_g)
    cp_t = pltpu.async_copy(t_hbm.at[pl.ds(base, _B_PER_W)], t_v, sem_t)
    cp_g.wait()
    cp_t.wait()
    def step(j, carry):
        base_j = j * (_UNROLL * _L)
        for u in range(_UNROLL):
            off = base_j + u * _L
            tv = t_v[pl.ds(off, _L)]
            r = (tv * _TIMESTEPS + _MAGIC) - _MAGIC
            idx = r.astype(jnp.int32)
            o_v[pl.ds(off, _L)] = plsc.load_gather(g_v, [idx])
        return carry

    lax.fori_loop(0, _B_PER_W // (_UNROLL * _L), step, 0)
    pltpu.sync_copy(o_v, out_hbm.at[pl.ds(base, _B_PER_W)])


@jax.jit
def kernel(t, gamma):
    mesh = plsc.VectorSubcoreMesh(
        core_axis_name="c", subcore_axis_name="s", num_cores=_NC
    )
    f = functools.partial(
        pl.kernel,
        mesh=mesh,
        out_type=jax.ShapeDtypeStruct((_BATCH,), jnp.float32),
        scratch_types=[
            pltpu.VMEM((_B_PER_W,), jnp.float32),
            pltpu.VMEM((_TABLE,), jnp.float32),
            pltpu.VMEM((_B_PER_W,), jnp.float32),
            pltpu.SemaphoreType.DMA,
            pltpu.SemaphoreType.DMA,
        ],
        compiler_params=pltpu.CompilerParams(
            needs_layout_passes=False,
            skip_device_barrier=True,
            disable_bounds_checks=True,
            disable_semaphore_checks=True,
        ),
    )(_body)
    return f(t, gamma)
